# Initial kernel scaffold; baseline (speedup 1.0000x reference)
import jax
import jax.numpy as jnp
from jax.experimental import pallas as pl


def kernel(x, edge_index, edge_attr, batch, global_features, W1, b1, We, Wc, bc):
    raise NotImplementedError("write your pallas kernel here")


# speedup vs baseline: 1.2039x; 1.2039x over previous
import jax
import jax.numpy as jnp
from jax.experimental import pallas as pl
from jax.experimental.pallas import tpu as pltpu

N = 10000
E = 320000
D = 128
DE = 4
G = 64
GF = 32
C = 10
SIGMA = 1.0

NB = 1000
NBLK = N // NB


def _h_kernel(x_ref, w_ref, b_ref, o_ref):
    o_ref[...] = jnp.dot(x_ref[...], w_ref[...],
                         preferred_element_type=jnp.float32) + b_ref[...]


def _compute_h(x, W1, b1):
    return pl.pallas_call(
        _h_kernel,
        grid=(NBLK,),
        in_specs=[
            pl.BlockSpec((NB, D), lambda i: (i, 0)),
            pl.BlockSpec((D, D), lambda i: (0, 0)),
            pl.BlockSpec((1, D), lambda i: (0, 0)),
        ],
        out_specs=pl.BlockSpec((NB, D), lambda i: (i, 0)),
        out_shape=jax.ShapeDtypeStruct((N, D), jnp.float32),
    )(x, W1, b1.reshape(1, D))


def _tail_kernel(agg_ref, h_ref, batch_ref, gfwc_ref, wc_ref, bc_ref,
                 o_ref, pooled_ref):
    i = pl.program_id(0)

    @pl.when(i == 0)
    def _init():
        pooled_ref[...] = jnp.zeros_like(pooled_ref)

    ge = jax.nn.relu(agg_ref[...] + h_ref[...])
    b = batch_ref[0, 0, :]
    seg = jax.lax.broadcasted_iota(jnp.int32, (G, NB), 0)
    onehot = (seg == b[None, :]).astype(jnp.float32)
    pooled_ref[...] += jnp.dot(onehot, ge, preferred_element_type=jnp.float32)

    @pl.when(i == NBLK - 1)
    def _final():
        o_ref[...] = (jnp.dot(pooled_ref[...], wc_ref[...],
                              preferred_element_type=jnp.float32)
                      + gfwc_ref[...] + bc_ref[...])


def _tail(agg, h, batch, global_features, Wc, bc):
    gf_wc = global_features @ Wc[D:, :]
    batch3 = batch.reshape(NBLK, 1, NB)
    return pl.pallas_call(
        _tail_kernel,
        grid=(NBLK,),
        in_specs=[
            pl.BlockSpec((NB, D), lambda i: (i, 0)),
            pl.BlockSpec((NB, D), lambda i: (i, 0)),
            pl.BlockSpec((1, 1, NB), lambda i: (i, 0, 0)),
            pl.BlockSpec((1, C), lambda i: (0, 0)),
            pl.BlockSpec((D, C), lambda i: (0, 0)),
            pl.BlockSpec((1, C), lambda i: (0, 0)),
        ],
        out_specs=pl.BlockSpec((G, C), lambda i: (0, 0)),
        out_shape=jax.ShapeDtypeStruct((G, C), jnp.float32),
        scratch_shapes=[pltpu.VMEM((G, D), jnp.float32)],
    )(agg, h, batch3, gf_wc, Wc[:D, :], bc.reshape(1, C))


def kernel(x, edge_index, edge_attr, batch, global_features, W1, b1, We, Wc, bc):
    src = edge_index[0]
    dst = edge_index[1]
    edge_batch = batch[src]
    min_vals = jax.ops.segment_min(edge_attr, edge_batch, num_segments=G)
    p = edge_attr - min_vals[edge_batch]
    edge_attr_exp = jnp.exp(-p ** 2 / (SIGMA ** 2 + 1e-06))

    h = _compute_h(x, W1, b1)

    gate = jax.nn.sigmoid(edge_attr_exp @ We)
    msg = h[src] * gate
    agg = jax.ops.segment_sum(msg, dst, num_segments=N)

    return _tail(agg, h, batch, global_features, Wc, bc)


# speedup vs baseline: 5.6446x; 4.6885x over previous
import functools

import jax
import jax.numpy as jnp
from jax import lax
from jax.experimental import pallas as pl
from jax.experimental.pallas import tpu as pltpu
from jax.experimental.pallas import tpu_sc as plsc

N = 10000
E = 320000
D = 128
DE = 4
G = 64
GF = 32
C = 10
INV = 1.0 / (1.0 + 1e-06)

NC = 2
NS = 16
L = 16
NW = NC * NS
EW = E // NW

ACH = 2000
ANCH = EW // ACH

_mesh = plsc.VectorSubcoreMesh(core_axis_name="c", subcore_axis_name="s")
_sc_params = pltpu.CompilerParams(needs_layout_passes=False)


@functools.partial(
    pl.kernel,
    out_type=(
        jax.ShapeDtypeStruct((E,), jnp.int32),
        jax.ShapeDtypeStruct((NW * G * DE,), jnp.float32),
    ),
    mesh=_mesh,
    scratch_types=[
        pltpu.VMEM((ACH,), jnp.int32),
        pltpu.VMEM((ACH,), jnp.int32),
        pltpu.VMEM((ACH * DE,), jnp.float32),
        pltpu.VMEM((L * G * DE,), jnp.float32),
        pltpu.VMEM((G * DE,), jnp.float32),
        pltpu.SemaphoreType.DMA,
    ],
    compiler_params=_sc_params,
)
def _sc_min_kernel(src_hbm, attr_hbm, batch_hbm, eb_hbm, minp_hbm,
                   src_v, eb_v, attr_v, tab_v, outb_v, sem):
    wid = lax.axis_index("s") * NC + lax.axis_index("c")
    lane = lax.iota(jnp.int32, 16)
    big = jnp.full((16,), 3.0e38, jnp.float32)

    def _init(j, _):
        tab_v[pl.ds(j * 16, 16)] = big
        return 0
    lax.fori_loop(0, L * G * DE // 16, _init, 0)

    for ci in range(ANCH):
        e0 = wid * EW + ci * ACH
        pltpu.sync_copy(src_hbm.at[pl.ds(e0, ACH)], src_v)
        pltpu.async_copy(batch_hbm.at[src_v], eb_v, sem).wait()
        pltpu.sync_copy(eb_v, eb_hbm.at[pl.ds(e0, ACH)])
        pltpu.sync_copy(attr_hbm.at[pl.ds(e0 * DE, ACH * DE)], attr_v)

        def _grp(i, _):
            eb16 = eb_v[pl.ds(i * 16, 16)]
            tbase = lane * (G * DE) + eb16 * DE
            abase = lane * DE + i * (16 * DE)
            for k in range(DE):
                a = plsc.load_gather(attr_v, [abase + k])
                t = tbase + k
                cur = plsc.load_gather(tab_v, [t])
                plsc.store_scatter(tab_v, [t], jnp.minimum(cur, a))
            return 0
        lax.fori_loop(0, ACH // 16, _grp, 0)

    for j in range(G * DE // 16):
        acc = tab_v[pl.ds(j * 16, 16)]
        for l in range(1, L):
            acc = jnp.minimum(acc, tab_v[pl.ds(l * (G * DE) + j * 16, 16)])
        outb_v[pl.ds(j * 16, 16)] = acc
    pltpu.sync_copy(outb_v, minp_hbm.at[pl.ds(wid * G * DE, G * DE)])


NB = 1000
NBLK = N // NB


def _h_body(x_ref, w_ref, b_ref, o_ref):
    o_ref[...] = jnp.dot(x_ref[...], w_ref[...],
                         preferred_element_type=jnp.float32) + b_ref[...]


def _compute_h(x, W1, b1):
    return pl.pallas_call(
        _h_body,
        grid=(NBLK,),
        in_specs=[
            pl.BlockSpec((NB, D), lambda i: (i, 0)),
            pl.BlockSpec((D, D), lambda i: (0, 0)),
            pl.BlockSpec((1, D), lambda i: (0, 0)),
        ],
        out_specs=pl.BlockSpec((NB, D), lambda i: (i, 0)),
        out_shape=jax.ShapeDtypeStruct((N, D), jnp.float32),
    )(x, W1, b1.reshape(1, D))


EB = 8000
EBLK = E // EB


def _gate_body(attr_ref, eb_ref, mv_ref, we_ref, o_ref):
    eb = eb_ref[0, 0, :]
    gid = lax.broadcasted_iota(jnp.int32, (EB, G), 1)
    onehot = (gid == eb[:, None]).astype(jnp.float32)
    mv_e = jnp.dot(onehot, mv_ref[...], preferred_element_type=jnp.float32)
    d = attr_ref[...] - mv_e
    ge = jnp.exp(-(d * d) * INV)
    gpre = jnp.dot(ge, we_ref[...], preferred_element_type=jnp.float32)
    o_ref[...] = jax.nn.sigmoid(gpre)


def _compute_gate(edge_attr, eb, min_vals, We):
    eb3 = eb.reshape(EBLK, 1, EB)
    return pl.pallas_call(
        _gate_body,
        grid=(EBLK,),
        in_specs=[
            pl.BlockSpec((EB, DE), lambda i: (i, 0)),
            pl.BlockSpec((1, 1, EB), lambda i: (i, 0, 0)),
            pl.BlockSpec((G, DE), lambda i: (0, 0)),
            pl.BlockSpec((DE, D), lambda i: (0, 0)),
        ],
        out_specs=pl.BlockSpec((EB, D), lambda i: (i, 0)),
        out_shape=jax.ShapeDtypeStruct((E, D), jnp.float32),
    )(edge_attr, eb3, min_vals, We)


BCH = 80
BNCH = EW // BCH
NPS = 624
NTAIL = N - NPS * NS


@functools.partial(
    pl.kernel,
    out_type=jax.ShapeDtypeStruct((NC, N, D), jnp.float32),
    mesh=_mesh,
    scratch_types=[
        pltpu.VMEM_SHARED((N, D), jnp.float32),
        pltpu.VMEM((BCH,), jnp.int32),
        pltpu.VMEM((BCH,), jnp.int32),
        pltpu.VMEM((BCH, D), jnp.float32),
        pltpu.VMEM((BCH, D), jnp.float32),
        pltpu.SemaphoreType.DMA,
    ],
    compiler_params=_sc_params,
)
def _sc_agg_kernel(src_hbm, dst_hbm, h_hbm, gate_hbm, agg_hbm,
                   aggS, src_v, dst_v, hrow_v, gate_v, sem):
    c = lax.axis_index("c")
    s = lax.axis_index("s")
    wid = s * NC + c

    z16 = jnp.zeros((16,), jnp.float32)

    def _z(e, _):
        for k in range(D // 16):
            hrow_v[e, pl.ds(k * 16, 16)] = z16
        return 0
    lax.fori_loop(0, BCH, _z, 0)
    for t in range(NPS // BCH):
        pltpu.sync_copy(hrow_v, aggS.at[pl.ds(s * NPS + t * BCH, BCH)])
    rem = NPS - (NPS // BCH) * BCH
    pltpu.sync_copy(hrow_v.at[pl.ds(0, rem)],
                    aggS.at[pl.ds(s * NPS + (NPS // BCH) * BCH, rem)])

    @pl.when(s == NS - 1)
    def _ztail():
        pltpu.sync_copy(hrow_v.at[pl.ds(0, NTAIL)],
                        aggS.at[pl.ds(NS * NPS, NTAIL)])
    plsc.subcore_barrier()

    def _chunk(ci, _):
        e0 = wid * EW + ci * BCH
        pltpu.sync_copy(src_hbm.at[pl.ds(e0, BCH)], src_v)
        pltpu.sync_copy(dst_hbm.at[pl.ds(e0, BCH)], dst_v)
        pltpu.sync_copy(gate_hbm.at[pl.ds(e0, BCH)], gate_v)
        pltpu.async_copy(h_hbm.at[src_v], hrow_v, sem).wait()

        def _mul(e, _):
            for k in range(D // 16):
                sl = pl.ds(k * 16, 16)
                hrow_v[e, sl] = hrow_v[e, sl] * gate_v[e, sl]
            return 0
        lax.fori_loop(0, BCH, _mul, 0)

        pltpu.sync_copy(hrow_v, aggS.at[dst_v], add=True)
        return 0
    lax.fori_loop(0, BNCH, _chunk, 0)

    plsc.subcore_barrier()
    pltpu.sync_copy(aggS.at[pl.ds(s * NPS, NPS)],
                    agg_hbm.at[c, pl.ds(s * NPS, NPS)])

    @pl.when(s == NS - 1)
    def _wtail():
        pltpu.sync_copy(aggS.at[pl.ds(NS * NPS, NTAIL)],
                        agg_hbm.at[c, pl.ds(NS * NPS, NTAIL)])


def _tail_body(agg0_ref, agg1_ref, h_ref, batch_ref, gfwc_ref, wc_ref, bc_ref,
               o_ref, pooled_ref):
    i = pl.program_id(0)

    @pl.when(i == 0)
    def _init():
        pooled_ref[...] = jnp.zeros_like(pooled_ref)

    ge = jax.nn.relu(agg0_ref[...] + agg1_ref[...] + h_ref[...])
    b = batch_ref[0, 0, :]
    seg = lax.broadcasted_iota(jnp.int32, (G, NB), 0)
    onehot = (seg == b[None, :]).astype(jnp.float32)
    pooled_ref[...] += jnp.dot(onehot, ge, preferred_element_type=jnp.float32)

    @pl.when(i == NBLK - 1)
    def _final():
        o_ref[...] = (jnp.dot(pooled_ref[...], wc_ref[...],
                              preferred_element_type=jnp.float32)
                      + gfwc_ref[...] + bc_ref[...])


def _tail(agg0, agg1, h, batch, global_features, Wc, bc):
    gf_wc = global_features @ Wc[D:, :]
    batch3 = batch.reshape(NBLK, 1, NB)
    return pl.pallas_call(
        _tail_body,
        grid=(NBLK,),
        in_specs=[
            pl.BlockSpec((NB, D), lambda i: (i, 0)),
            pl.BlockSpec((NB, D), lambda i: (i, 0)),
            pl.BlockSpec((NB, D), lambda i: (i, 0)),
            pl.BlockSpec((1, 1, NB), lambda i: (i, 0, 0)),
            pl.BlockSpec((1, C), lambda i: (0, 0)),
            pl.BlockSpec((D, C), lambda i: (0, 0)),
            pl.BlockSpec((1, C), lambda i: (0, 0)),
        ],
        out_specs=pl.BlockSpec((G, C), lambda i: (0, 0)),
        out_shape=jax.ShapeDtypeStruct((G, C), jnp.float32),
        scratch_shapes=[pltpu.VMEM((G, D), jnp.float32)],
    )(agg0, agg1, h, batch3, gf_wc, Wc[:D, :], bc.reshape(1, C))


def kernel(x, edge_index, edge_attr, batch, global_features, W1, b1, We, Wc, bc):
    src = edge_index[0]
    dst = edge_index[1]
    attr_flat = edge_attr.reshape(E * DE)

    eb, minp = _sc_min_kernel(src, attr_flat, batch)
    min_vals = jnp.min(minp.reshape(NW, G * DE), axis=0).reshape(G, DE)

    h = _compute_h(x, W1, b1)
    gate = _compute_gate(edge_attr, eb, min_vals, We)

    agg2 = _sc_agg_kernel(src, dst, h, gate)

    return _tail(agg2[0], agg2[1], h, batch, global_features, Wc, bc)


# speedup vs baseline: 7.4816x; 1.3254x over previous
import functools

import jax
import jax.numpy as jnp
from jax import lax
from jax.experimental import pallas as pl
from jax.experimental.pallas import tpu as pltpu
from jax.experimental.pallas import tpu_sc as plsc

N = 10000
E = 320000
D = 128
DE = 4
G = 64
GF = 32
C = 10
INV = 1.0 / (1.0 + 1e-06)

NC = 2
NS = 16
L = 16
NW = NC * NS
EW = E // NW

ACH = 2000
ANCH = EW // ACH

_mesh = plsc.VectorSubcoreMesh(core_axis_name="c", subcore_axis_name="s")
_sc_params = pltpu.CompilerParams(needs_layout_passes=False)


@functools.partial(
    pl.kernel,
    out_type=(
        jax.ShapeDtypeStruct((E,), jnp.int32),
        jax.ShapeDtypeStruct((NW * G * DE,), jnp.float32),
    ),
    mesh=_mesh,
    scratch_types=[
        pltpu.VMEM((ACH,), jnp.int32),
        pltpu.VMEM((ACH,), jnp.int32),
        pltpu.VMEM((ACH * DE,), jnp.float32),
        pltpu.VMEM((L * G * DE,), jnp.float32),
        pltpu.VMEM((G * DE,), jnp.float32),
        pltpu.SemaphoreType.DMA,
    ],
    compiler_params=_sc_params,
)
def _sc_min_kernel(src_hbm, attr_hbm, batch_hbm, eb_hbm, minp_hbm,
                   src_v, eb_v, attr_v, tab_v, outb_v, sem):
    wid = lax.axis_index("s") * NC + lax.axis_index("c")
    lane = lax.iota(jnp.int32, 16)
    big = jnp.full((16,), 3.0e38, jnp.float32)

    def _init(j, _):
        tab_v[pl.ds(j * 16, 16)] = big
        return 0
    lax.fori_loop(0, L * G * DE // 16, _init, 0)

    for ci in range(ANCH):
        e0 = wid * EW + ci * ACH
        pltpu.sync_copy(src_hbm.at[pl.ds(e0, ACH)], src_v)
        pltpu.async_copy(batch_hbm.at[src_v], eb_v, sem).wait()
        pltpu.sync_copy(eb_v, eb_hbm.at[pl.ds(e0, ACH)])
        pltpu.sync_copy(attr_hbm.at[pl.ds(e0 * DE, ACH * DE)], attr_v)

        def _grp(i, _):
            eb16 = eb_v[pl.ds(i * 16, 16)]
            tbase = lane * (G * DE) + eb16 * DE
            abase = lane * DE + i * (16 * DE)
            for k in range(DE):
                a = plsc.load_gather(attr_v, [abase + k])
                t = tbase + k
                cur = plsc.load_gather(tab_v, [t])
                plsc.store_scatter(tab_v, [t], jnp.minimum(cur, a))
            return 0
        lax.fori_loop(0, ACH // 16, _grp, 0)

    for j in range(G * DE // 16):
        acc = tab_v[pl.ds(j * 16, 16)]
        for l in range(1, L):
            acc = jnp.minimum(acc, tab_v[pl.ds(l * (G * DE) + j * 16, 16)])
        outb_v[pl.ds(j * 16, 16)] = acc
    pltpu.sync_copy(outb_v, minp_hbm.at[pl.ds(wid * G * DE, G * DE)])


NB = 1000
NBLK = N // NB


def _h_body(x_ref, w_ref, b_ref, o_ref):
    o_ref[...] = jnp.dot(x_ref[...], w_ref[...],
                         preferred_element_type=jnp.float32) + b_ref[...]


def _compute_h(x, W1, b1):
    return pl.pallas_call(
        _h_body,
        grid=(NBLK,),
        in_specs=[
            pl.BlockSpec((NB, D), lambda i: (i, 0)),
            pl.BlockSpec((D, D), lambda i: (0, 0)),
            pl.BlockSpec((1, D), lambda i: (0, 0)),
        ],
        out_specs=pl.BlockSpec((NB, D), lambda i: (i, 0)),
        out_shape=jax.ShapeDtypeStruct((N, D), jnp.float32),
    )(x, W1, b1.reshape(1, D))


EB = 8000
EBLK = E // EB


def _gate_body(attr_ref, eb_ref, mv_ref, we_ref, o_ref):
    eb = eb_ref[0, 0, :]
    gid = lax.broadcasted_iota(jnp.int32, (EB, G), 1)
    onehot = (gid == eb[:, None]).astype(jnp.float32)
    mv_e = jnp.dot(onehot, mv_ref[...], preferred_element_type=jnp.float32)
    d = attr_ref[...] - mv_e
    ge = jnp.exp(-(d * d) * INV)
    gpre = jnp.dot(ge, we_ref[...], preferred_element_type=jnp.float32)
    o_ref[...] = jax.nn.sigmoid(gpre)


def _compute_gate(edge_attr, eb, min_vals, We):
    eb3 = eb.reshape(EBLK, 1, EB)
    return pl.pallas_call(
        _gate_body,
        grid=(EBLK,),
        in_specs=[
            pl.BlockSpec((EB, DE), lambda i: (i, 0)),
            pl.BlockSpec((1, 1, EB), lambda i: (i, 0, 0)),
            pl.BlockSpec((G, DE), lambda i: (0, 0)),
            pl.BlockSpec((DE, D), lambda i: (0, 0)),
        ],
        out_specs=pl.BlockSpec((EB, D), lambda i: (i, 0)),
        out_shape=jax.ShapeDtypeStruct((E, D), jnp.float32),
    )(edge_attr, eb3, min_vals, We)


BCH = 80
BNCH = EW // BCH
NPS = 624
NTAIL = N - NPS * NS


@functools.partial(
    pl.kernel,
    out_type=jax.ShapeDtypeStruct((NC, N, D), jnp.float32),
    mesh=_mesh,
    scratch_types=[
        pltpu.VMEM_SHARED((N, D), jnp.float32),
        pltpu.VMEM((BCH,), jnp.int32),
        pltpu.VMEM((BCH,), jnp.int32),
        pltpu.VMEM((BCH, D), jnp.float32),
        pltpu.VMEM((BCH, D), jnp.float32),
        pltpu.VMEM((BCH,), jnp.int32),
        pltpu.VMEM((BCH,), jnp.int32),
        pltpu.VMEM((BCH, D), jnp.float32),
        pltpu.VMEM((BCH, D), jnp.float32),
        pltpu.SemaphoreType.DMA,
        pltpu.SemaphoreType.DMA,
        pltpu.SemaphoreType.DMA,
        pltpu.SemaphoreType.DMA,
    ],
    compiler_params=_sc_params,
)
def _sc_agg_kernel(src_hbm, dst_hbm, h_hbm, gate_hbm, agg_hbm,
                   aggS, src0, dst0, hrow0, gate0, src1, dst1, hrow1, gate1,
                   asem0, asem1, gsem0, gsem1):
    c = lax.axis_index("c")
    s = lax.axis_index("s")
    wid = s * NC + c
    bufs = ((src0, dst0, hrow0, gate0, asem0, gsem0),
            (src1, dst1, hrow1, gate1, asem1, gsem1))

    z16 = jnp.zeros((16,), jnp.float32)

    def _z(e, _):
        for k in range(D // 16):
            hrow0[e, pl.ds(k * 16, 16)] = z16
        return 0
    lax.fori_loop(0, BCH, _z, 0)
    for t in range(NPS // BCH):
        pltpu.sync_copy(hrow0, aggS.at[pl.ds(s * NPS + t * BCH, BCH)])
    rem = NPS - (NPS // BCH) * BCH
    pltpu.sync_copy(hrow0.at[pl.ds(0, rem)],
                    aggS.at[pl.ds(s * NPS + (NPS // BCH) * BCH, rem)])

    @pl.when(s == NS - 1)
    def _ztail():
        pltpu.sync_copy(hrow0.at[pl.ds(0, NTAIL)],
                        aggS.at[pl.ds(NS * NPS, NTAIL)])
    plsc.subcore_barrier()

    def _startA(b, ci):
        sv, dv, _, gv, asem, _ = bufs[b]
        e0 = wid * EW + ci * BCH
        pltpu.async_copy(src_hbm.at[pl.ds(e0, BCH)], sv, asem)
        pltpu.async_copy(dst_hbm.at[pl.ds(e0, BCH)], dv, asem)
        pltpu.async_copy(gate_hbm.at[pl.ds(e0, BCH)], gv, asem)

    def _waitA(b):
        sv, dv, _, gv, asem, _ = bufs[b]
        pltpu.make_async_copy(src_hbm.at[pl.ds(0, BCH)], sv, asem).wait()
        pltpu.make_async_copy(dst_hbm.at[pl.ds(0, BCH)], dv, asem).wait()
        pltpu.make_async_copy(gate_hbm.at[pl.ds(0, BCH)], gv, asem).wait()

    def _startG(b):
        sv, _, hv, _, _, gsem = bufs[b]
        pltpu.async_copy(h_hbm.at[sv], hv, gsem)

    def _waitG(b):
        sv, _, hv, _, _, gsem = bufs[b]
        pltpu.make_async_copy(h_hbm.at[sv], hv, gsem).wait()

    def _compute_scatter(b):
        _, dv, hv, gv, _, _ = bufs[b]

        def _mul(e, _):
            for k in range(D // 16):
                sl = pl.ds(k * 16, 16)
                hv[e, sl] = hv[e, sl] * gv[e, sl]
            return 0
        lax.fori_loop(0, BCH, _mul, 0)
        pltpu.sync_copy(hv, aggS.at[dv], add=True)

    _startA(0, 0)
    _startA(1, 1)
    _waitA(0)
    _startG(0)

    def _pair(p, _):
        for b in range(2):
            ci = 2 * p + b

            @pl.when(ci + 1 < BNCH)
            def _nxt():
                _waitA(1 - b)
                _startG(1 - b)
            _waitG(b)
            _compute_scatter(b)

            @pl.when(ci + 2 < BNCH)
            def _pref():
                _startA(b, ci + 2)
        return 0
    lax.fori_loop(0, BNCH // 2, _pair, 0)

    _waitG(0)
    _compute_scatter(0)

    plsc.subcore_barrier()
    pltpu.sync_copy(aggS.at[pl.ds(s * NPS, NPS)],
                    agg_hbm.at[c, pl.ds(s * NPS, NPS)])

    @pl.when(s == NS - 1)
    def _wtail():
        pltpu.sync_copy(aggS.at[pl.ds(NS * NPS, NTAIL)],
                        agg_hbm.at[c, pl.ds(NS * NPS, NTAIL)])


def _tail_body(agg0_ref, agg1_ref, h_ref, batch_ref, gfwc_ref, wc_ref, bc_ref,
               o_ref, pooled_ref):
    i = pl.program_id(0)

    @pl.when(i == 0)
    def _init():
        pooled_ref[...] = jnp.zeros_like(pooled_ref)

    ge = jax.nn.relu(agg0_ref[...] + agg1_ref[...] + h_ref[...])
    b = batch_ref[0, 0, :]
    seg = lax.broadcasted_iota(jnp.int32, (G, NB), 0)
    onehot = (seg == b[None, :]).astype(jnp.float32)
    pooled_ref[...] += jnp.dot(onehot, ge, preferred_element_type=jnp.float32)

    @pl.when(i == NBLK - 1)
    def _final():
        o_ref[...] = (jnp.dot(pooled_ref[...], wc_ref[...],
                              preferred_element_type=jnp.float32)
                      + gfwc_ref[...] + bc_ref[...])


def _tail(agg0, agg1, h, batch, global_features, Wc, bc):
    gf_wc = global_features @ Wc[D:, :]
    batch3 = batch.reshape(NBLK, 1, NB)
    return pl.pallas_call(
        _tail_body,
        grid=(NBLK,),
        in_specs=[
            pl.BlockSpec((NB, D), lambda i: (i, 0)),
            pl.BlockSpec((NB, D), lambda i: (i, 0)),
            pl.BlockSpec((NB, D), lambda i: (i, 0)),
            pl.BlockSpec((1, 1, NB), lambda i: (i, 0, 0)),
            pl.BlockSpec((1, C), lambda i: (0, 0)),
            pl.BlockSpec((D, C), lambda i: (0, 0)),
            pl.BlockSpec((1, C), lambda i: (0, 0)),
        ],
        out_specs=pl.BlockSpec((G, C), lambda i: (0, 0)),
        out_shape=jax.ShapeDtypeStruct((G, C), jnp.float32),
        scratch_shapes=[pltpu.VMEM((G, D), jnp.float32)],
    )(agg0, agg1, h, batch3, gf_wc, Wc[:D, :], bc.reshape(1, C))


def kernel(x, edge_index, edge_attr, batch, global_features, W1, b1, We, Wc, bc):
    src = edge_index[0]
    dst = edge_index[1]
    attr_flat = edge_attr.reshape(E * DE)

    eb, minp = _sc_min_kernel(src, attr_flat, batch)
    min_vals = jnp.min(minp.reshape(NW, G * DE), axis=0).reshape(G, DE)

    h = _compute_h(x, W1, b1)
    gate = _compute_gate(edge_attr, eb, min_vals, We)

    agg2 = _sc_agg_kernel(src, dst, h, gate)

    return _tail(agg2[0], agg2[1], h, batch, global_features, Wc, bc)


# speedup vs baseline: 8.1243x; 1.0859x over previous
import functools

import jax
import jax.numpy as jnp
from jax import lax
from jax.experimental import pallas as pl
from jax.experimental.pallas import tpu as pltpu
from jax.experimental.pallas import tpu_sc as plsc

N = 10000
E = 320000
D = 128
DE = 4
G = 64
GF = 32
C = 10
INV = 1.0 / (1.0 + 1e-06)

NC = 2
NS = 16
L = 16
NW = NC * NS
EW = E // NW

ACH = 2000
ANCH = EW // ACH

_mesh = plsc.VectorSubcoreMesh(core_axis_name="c", subcore_axis_name="s")
_sc_params = pltpu.CompilerParams(needs_layout_passes=False)


@functools.partial(
    pl.kernel,
    out_type=(
        jax.ShapeDtypeStruct((E,), jnp.int32),
        jax.ShapeDtypeStruct((NW * G * DE,), jnp.float32),
    ),
    mesh=_mesh,
    scratch_types=[
        pltpu.VMEM((ACH,), jnp.int32),
        pltpu.VMEM((ACH,), jnp.int32),
        pltpu.VMEM((ACH * DE,), jnp.float32),
        pltpu.VMEM((L * G * DE,), jnp.float32),
        pltpu.VMEM((G * DE,), jnp.float32),
        pltpu.SemaphoreType.DMA,
    ],
    compiler_params=_sc_params,
)
def _sc_min_kernel(src_hbm, attr_hbm, batch_hbm, eb_hbm, minp_hbm,
                   src_v, eb_v, attr_v, tab_v, outb_v, sem):
    wid = lax.axis_index("s") * NC + lax.axis_index("c")
    lane = lax.iota(jnp.int32, 16)
    big = jnp.full((16,), 3.0e38, jnp.float32)

    def _init(j, _):
        tab_v[pl.ds(j * 16, 16)] = big
        return 0
    lax.fori_loop(0, L * G * DE // 16, _init, 0)

    for ci in range(ANCH):
        e0 = wid * EW + ci * ACH
        pltpu.sync_copy(src_hbm.at[pl.ds(e0, ACH)], src_v)
        pltpu.async_copy(batch_hbm.at[src_v], eb_v, sem).wait()
        pltpu.sync_copy(eb_v, eb_hbm.at[pl.ds(e0, ACH)])
        pltpu.sync_copy(attr_hbm.at[pl.ds(e0 * DE, ACH * DE)], attr_v)

        def _grp(i, _):
            eb16 = eb_v[pl.ds(i * 16, 16)]
            tbase = lane * (G * DE) + eb16 * DE
            abase = lane * DE + i * (16 * DE)
            for k in range(DE):
                a = plsc.load_gather(attr_v, [abase + k])
                t = tbase + k
                cur = plsc.load_gather(tab_v, [t])
                plsc.store_scatter(tab_v, [t], jnp.minimum(cur, a))
            return 0
        lax.fori_loop(0, ACH // 16, _grp, 0)

    for j in range(G * DE // 16):
        acc = tab_v[pl.ds(j * 16, 16)]
        for l in range(1, L):
            acc = jnp.minimum(acc, tab_v[pl.ds(l * (G * DE) + j * 16, 16)])
        outb_v[pl.ds(j * 16, 16)] = acc
    pltpu.sync_copy(outb_v, minp_hbm.at[pl.ds(wid * G * DE, G * DE)])


NB = 1000
NBLK = N // NB


def _h_body(x_ref, w_ref, b_ref, o_ref):
    o_ref[...] = jnp.dot(x_ref[...], w_ref[...],
                         preferred_element_type=jnp.float32) + b_ref[...]


def _compute_h(x, W1, b1):
    return pl.pallas_call(
        _h_body,
        grid=(NBLK,),
        in_specs=[
            pl.BlockSpec((NB, D), lambda i: (i, 0)),
            pl.BlockSpec((D, D), lambda i: (0, 0)),
            pl.BlockSpec((1, D), lambda i: (0, 0)),
        ],
        out_specs=pl.BlockSpec((NB, D), lambda i: (i, 0)),
        out_shape=jax.ShapeDtypeStruct((N, D), jnp.float32),
    )(x, W1, b1.reshape(1, D))


EB = 8000
EBLK = E // EB


def _gate_body(attr_ref, eb_ref, minp_ref, we_ref, o_ref):
    eb = eb_ref[0, 0, :]
    gid = lax.broadcasted_iota(jnp.int32, (EB, G), 1)
    onehot = (gid == eb[:, None]).astype(jnp.float32)
    mv = jnp.min(minp_ref[...], axis=0)
    mv_e = jnp.dot(onehot, mv, preferred_element_type=jnp.float32)
    d = attr_ref[...] - mv_e
    ge = jnp.exp(-(d * d) * INV)
    gpre = jnp.dot(ge, we_ref[...], preferred_element_type=jnp.float32)
    o_ref[...] = jax.nn.sigmoid(gpre)


def _compute_gate(edge_attr, eb, minp, We):
    eb3 = eb.reshape(EBLK, 1, EB)
    minp3 = minp.reshape(NW, G, DE)
    return pl.pallas_call(
        _gate_body,
        grid=(EBLK,),
        in_specs=[
            pl.BlockSpec((EB, DE), lambda i: (i, 0)),
            pl.BlockSpec((1, 1, EB), lambda i: (i, 0, 0)),
            pl.BlockSpec((NW, G, DE), lambda i: (0, 0, 0)),
            pl.BlockSpec((DE, D), lambda i: (0, 0)),
        ],
        out_specs=pl.BlockSpec((EB, D), lambda i: (i, 0)),
        out_shape=jax.ShapeDtypeStruct((E, D), jnp.float32),
    )(edge_attr, eb3, minp3, We)


BCH = 80
BNCH = EW // BCH
NPS = 624
NTAIL = N - NPS * NS


@functools.partial(
    pl.kernel,
    out_type=jax.ShapeDtypeStruct((NC, N, D), jnp.float32),
    mesh=_mesh,
    scratch_types=[
        pltpu.VMEM_SHARED((N, D), jnp.float32),
        pltpu.VMEM((BCH,), jnp.int32),
        pltpu.VMEM((BCH,), jnp.int32),
        pltpu.VMEM((BCH, D), jnp.float32),
        pltpu.VMEM((BCH, D), jnp.float32),
        pltpu.VMEM((BCH,), jnp.int32),
        pltpu.VMEM((BCH,), jnp.int32),
        pltpu.VMEM((BCH, D), jnp.float32),
        pltpu.VMEM((BCH, D), jnp.float32),
        pltpu.SemaphoreType.DMA,
        pltpu.SemaphoreType.DMA,
        pltpu.SemaphoreType.DMA,
        pltpu.SemaphoreType.DMA,
        pltpu.SemaphoreType.DMA,
        pltpu.SemaphoreType.DMA,
    ],
    compiler_params=_sc_params,
)
def _sc_agg_kernel(src_hbm, dst_hbm, h_hbm, gate_hbm, agg_hbm,
                   aggS, src0, dst0, hrow0, gate0, src1, dst1, hrow1, gate1,
                   asem0, asem1, gsem0, gsem1, ssem0, ssem1):
    c = lax.axis_index("c")
    s = lax.axis_index("s")
    wid = s * NC + c
    bufs = ((src0, dst0, hrow0, gate0, asem0, gsem0, ssem0),
            (src1, dst1, hrow1, gate1, asem1, gsem1, ssem1))

    z16 = jnp.zeros((16,), jnp.float32)

    def _z(e, _):
        for k in range(D // 16):
            hrow0[e, pl.ds(k * 16, 16)] = z16
        return 0
    lax.fori_loop(0, BCH, _z, 0)
    for t in range(NPS // BCH):
        pltpu.sync_copy(hrow0, aggS.at[pl.ds(s * NPS + t * BCH, BCH)])
    rem = NPS - (NPS // BCH) * BCH
    pltpu.sync_copy(hrow0.at[pl.ds(0, rem)],
                    aggS.at[pl.ds(s * NPS + (NPS // BCH) * BCH, rem)])

    @pl.when(s == NS - 1)
    def _ztail():
        pltpu.sync_copy(hrow0.at[pl.ds(0, NTAIL)],
                        aggS.at[pl.ds(NS * NPS, NTAIL)])
    plsc.subcore_barrier()

    def _startA(b, ci):
        sv, dv, _, gv, asem, _, _ = bufs[b]
        e0 = wid * EW + ci * BCH
        pltpu.async_copy(src_hbm.at[pl.ds(e0, BCH)], sv, asem)
        pltpu.async_copy(dst_hbm.at[pl.ds(e0, BCH)], dv, asem)
        pltpu.async_copy(gate_hbm.at[pl.ds(e0, BCH)], gv, asem)

    def _waitA(b):
        sv, dv, _, gv, asem, _, _ = bufs[b]
        pltpu.make_async_copy(src_hbm.at[pl.ds(0, BCH)], sv, asem).wait()
        pltpu.make_async_copy(dst_hbm.at[pl.ds(0, BCH)], dv, asem).wait()
        pltpu.make_async_copy(gate_hbm.at[pl.ds(0, BCH)], gv, asem).wait()

    def _startG(b):
        sv, _, hv, _, _, gsem, _ = bufs[b]
        pltpu.async_copy(h_hbm.at[sv], hv, gsem)

    def _waitG(b):
        sv, _, hv, _, _, gsem, _ = bufs[b]
        pltpu.make_async_copy(h_hbm.at[sv], hv, gsem).wait()

    def _compute(b):
        _, dv, hv, gv, _, _, ssem = bufs[b]

        def _mul(e2, _):
            for u in range(2):
                for k in range(D // 16):
                    sl = pl.ds(k * 16, 16)
                    hv[e2 * 2 + u, sl] = hv[e2 * 2 + u, sl] * gv[e2 * 2 + u, sl]
            return 0
        lax.fori_loop(0, BCH // 2, _mul, 0)
        pltpu.async_copy(hv, aggS.at[dv], ssem, add=True)

    def _waitS(b):
        _, dv, hv, _, _, _, ssem = bufs[b]
        pltpu.make_async_copy(hv, aggS.at[dv], ssem).wait()

    _startA(0, 0)
    _startA(1, 1)
    _waitA(0)
    _startG(0)

    def _pair(p, _):
        for b in range(2):
            ci = 2 * p + b

            @pl.when(ci + 1 < BNCH)
            def _nxt():
                _waitA(1 - b)

                @pl.when(ci >= 1)
                def _ws():
                    _waitS(1 - b)
                _startG(1 - b)
            _waitG(b)
            _compute(b)

            @pl.when(ci + 2 < BNCH)
            def _pref():
                _startA(b, ci + 2)
        return 0
    lax.fori_loop(0, BNCH // 2, _pair, 0)

    _waitG(0)
    _compute(0)
    _waitS(1)
    _waitS(0)

    plsc.subcore_barrier()
    pltpu.sync_copy(aggS.at[pl.ds(s * NPS, NPS)],
                    agg_hbm.at[c, pl.ds(s * NPS, NPS)])

    @pl.when(s == NS - 1)
    def _wtail():
        pltpu.sync_copy(aggS.at[pl.ds(NS * NPS, NTAIL)],
                        agg_hbm.at[c, pl.ds(NS * NPS, NTAIL)])


def _tail_body(agg0_ref, agg1_ref, h_ref, batch_ref, gf_ref, wc_ref, bc_ref,
               o_ref, pooled_ref):
    i = pl.program_id(0)

    @pl.when(i == 0)
    def _init():
        pooled_ref[...] = jnp.zeros_like(pooled_ref)

    ge = jax.nn.relu(agg0_ref[0] + agg1_ref[0] + h_ref[...])
    b = batch_ref[0, 0, :]
    seg = lax.broadcasted_iota(jnp.int32, (G, NB), 0)
    onehot = (seg == b[None, :]).astype(jnp.float32)
    pooled_ref[...] += jnp.dot(onehot, ge, preferred_element_type=jnp.float32)

    @pl.when(i == NBLK - 1)
    def _final():
        gf_wc = jnp.dot(gf_ref[...], wc_ref[D:, :],
                        preferred_element_type=jnp.float32)
        o_ref[...] = (jnp.dot(pooled_ref[...], wc_ref[:D, :],
                              preferred_element_type=jnp.float32)
                      + gf_wc + bc_ref[...])


def _tail(agg2, h, batch, global_features, Wc, bc):
    batch3 = batch.reshape(NBLK, 1, NB)
    return pl.pallas_call(
        _tail_body,
        grid=(NBLK,),
        in_specs=[
            pl.BlockSpec((1, NB, D), lambda i: (0, i, 0)),
            pl.BlockSpec((1, NB, D), lambda i: (1, i, 0)),
            pl.BlockSpec((NB, D), lambda i: (i, 0)),
            pl.BlockSpec((1, 1, NB), lambda i: (i, 0, 0)),
            pl.BlockSpec((1, GF), lambda i: (0, 0)),
            pl.BlockSpec((D + GF, C), lambda i: (0, 0)),
            pl.BlockSpec((1, C), lambda i: (0, 0)),
        ],
        out_specs=pl.BlockSpec((G, C), lambda i: (0, 0)),
        out_shape=jax.ShapeDtypeStruct((G, C), jnp.float32),
        scratch_shapes=[pltpu.VMEM((G, D), jnp.float32)],
    )(agg2, agg2, h, batch3, global_features, Wc, bc.reshape(1, C))


def kernel(x, edge_index, edge_attr, batch, global_features, W1, b1, We, Wc, bc):
    src = edge_index[0]
    dst = edge_index[1]
    attr_flat = edge_attr.reshape(E * DE)

    eb, minp = _sc_min_kernel(src, attr_flat, batch)

    h = _compute_h(x, W1, b1)
    gate = _compute_gate(edge_attr, eb, minp, We)

    agg2 = _sc_agg_kernel(src, dst, h, gate)

    return _tail(agg2, h, batch, global_features, Wc, bc)


# speedup vs baseline: 8.3301x; 1.0253x over previous
import functools

import jax
import jax.numpy as jnp
from jax import lax
from jax.experimental import pallas as pl
from jax.experimental.pallas import tpu as pltpu
from jax.experimental.pallas import tpu_sc as plsc

N = 10000
E = 320000
D = 128
DE = 4
G = 64
GF = 32
C = 10
INV = 1.0 / (1.0 + 1e-06)

NC = 2
NS = 16
L = 16
NW = NC * NS
EW = E // NW

ACH = 2000
ANCH = EW // ACH

_mesh = plsc.VectorSubcoreMesh(core_axis_name="c", subcore_axis_name="s")
_sc_params = pltpu.CompilerParams(needs_layout_passes=False)


@functools.partial(
    pl.kernel,
    out_type=(
        jax.ShapeDtypeStruct((E,), jnp.int32),
        jax.ShapeDtypeStruct((NW * G * DE,), jnp.float32),
    ),
    mesh=_mesh,
    scratch_types=[
        pltpu.VMEM((ACH,), jnp.int32),
        pltpu.VMEM((ACH,), jnp.int32),
        pltpu.VMEM((ACH * DE,), jnp.float32),
        pltpu.VMEM((ACH,), jnp.int32),
        pltpu.VMEM((ACH,), jnp.int32),
        pltpu.VMEM((ACH * DE,), jnp.float32),
        pltpu.VMEM((L * G * DE,), jnp.float32),
        pltpu.VMEM((G * DE,), jnp.float32),
        pltpu.SemaphoreType.DMA,
        pltpu.SemaphoreType.DMA,
        pltpu.SemaphoreType.DMA,
        pltpu.SemaphoreType.DMA,
    ],
    compiler_params=_sc_params,
)
def _sc_min_kernel(src_hbm, attr_hbm, batch_hbm, eb_hbm, minp_hbm,
                   asrc0, aeb0, aattr0, asrc1, aeb1, aattr1,
                   tab_v, outb_v, lsem0, lsem1, gsem0, gsem1):
    wid = lax.axis_index("s") * NC + lax.axis_index("c")
    lane = lax.iota(jnp.int32, 16)
    big = jnp.full((16,), 3.0e38, jnp.float32)
    abufs = ((asrc0, aeb0, aattr0, lsem0, gsem0),
             (asrc1, aeb1, aattr1, lsem1, gsem1))

    def _init(j, _):
        tab_v[pl.ds(j * 16, 16)] = big
        return 0
    lax.fori_loop(0, L * G * DE // 16, _init, 0)

    def _sL(b, ci):
        sv, _, av, lsem, _ = abufs[b]
        e0 = wid * EW + ci * ACH
        pltpu.async_copy(src_hbm.at[pl.ds(e0, ACH)], sv, lsem)
        pltpu.async_copy(attr_hbm.at[pl.ds(e0 * DE, ACH * DE)], av, lsem)

    def _wL(b):
        sv, _, av, lsem, _ = abufs[b]
        pltpu.make_async_copy(src_hbm.at[pl.ds(0, ACH)], sv, lsem).wait()
        pltpu.make_async_copy(attr_hbm.at[pl.ds(0, ACH * DE)], av, lsem).wait()

    def _sG(b):
        sv, ev, _, _, gsem = abufs[b]
        pltpu.async_copy(batch_hbm.at[sv], ev, gsem)

    def _wG(b):
        sv, ev, _, _, gsem = abufs[b]
        pltpu.make_async_copy(batch_hbm.at[sv], ev, gsem).wait()

    def _acompute(b, ci):
        sv, ev, av, _, _ = abufs[b]
        e0 = wid * EW + ci * ACH

        def _grp(i, _):
            eb16 = ev[pl.ds(i * 16, 16)]
            tbase = lane * (G * DE) + eb16 * DE
            abase = lane * DE + i * (16 * DE)
            for k in range(DE):
                a = plsc.load_gather(av, [abase + k])
                t = tbase + k
                cur = plsc.load_gather(tab_v, [t])
                plsc.store_scatter(tab_v, [t], jnp.minimum(cur, a))
            return 0
        lax.fori_loop(0, ACH // 16, _grp, 0)
        pltpu.sync_copy(ev, eb_hbm.at[pl.ds(e0, ACH)])

    _sL(0, 0)
    _sL(1, 1)
    _wL(0)
    _sG(0)
    for ci in range(ANCH):
        b = ci % 2
        if ci + 1 < ANCH:
            _wL(1 - b)
            _sG(1 - b)
        _wG(b)
        _acompute(b, ci)
        if ci + 2 < ANCH:
            _sL(b, ci + 2)

    for j in range(G * DE // 16):
        acc = tab_v[pl.ds(j * 16, 16)]
        for l in range(1, L):
            acc = jnp.minimum(acc, tab_v[pl.ds(l * (G * DE) + j * 16, 16)])
        outb_v[pl.ds(j * 16, 16)] = acc
    pltpu.sync_copy(outb_v, minp_hbm.at[pl.ds(wid * G * DE, G * DE)])


NB = 1000
NBLK = N // NB


def _h_body(x_ref, w_ref, b_ref, o_ref):
    o_ref[...] = jnp.dot(x_ref[...], w_ref[...],
                         preferred_element_type=jnp.float32) + b_ref[...]


def _compute_h(x, W1, b1):
    return pl.pallas_call(
        _h_body,
        grid=(NBLK,),
        in_specs=[
            pl.BlockSpec((NB, D), lambda i: (i, 0)),
            pl.BlockSpec((D, D), lambda i: (0, 0)),
            pl.BlockSpec((1, D), lambda i: (0, 0)),
        ],
        out_specs=pl.BlockSpec((NB, D), lambda i: (i, 0)),
        out_shape=jax.ShapeDtypeStruct((N, D), jnp.float32),
    )(x, W1, b1.reshape(1, D))


EB = 8000
EBLK = E // EB


def _gate_body(attr_ref, eb_ref, minp_ref, we_ref, o_ref):
    eb = eb_ref[0, 0, :]
    gid = lax.broadcasted_iota(jnp.int32, (EB, G), 1)
    onehot = (gid == eb[:, None]).astype(jnp.float32)
    mv = jnp.min(minp_ref[...], axis=0)
    mv_e = jnp.dot(onehot, mv, preferred_element_type=jnp.float32)
    d = attr_ref[...] - mv_e
    ge = jnp.exp(-(d * d) * INV)
    gpre = jnp.dot(ge, we_ref[...], preferred_element_type=jnp.float32)
    o_ref[...] = 1.0 / (1.0 + jnp.exp(-gpre))


def _compute_gate(edge_attr, eb, minp, We):
    eb3 = eb.reshape(EBLK, 1, EB)
    minp3 = minp.reshape(NW, G, DE)
    return pl.pallas_call(
        _gate_body,
        grid=(EBLK,),
        in_specs=[
            pl.BlockSpec((EB, DE), lambda i: (i, 0)),
            pl.BlockSpec((1, 1, EB), lambda i: (i, 0, 0)),
            pl.BlockSpec((NW, G, DE), lambda i: (0, 0, 0)),
            pl.BlockSpec((DE, D), lambda i: (0, 0)),
        ],
        out_specs=pl.BlockSpec((EB, D), lambda i: (i, 0)),
        out_shape=jax.ShapeDtypeStruct((E, D), jnp.float32),
    )(edge_attr, eb3, minp3, We)


BCH = 80
BNCH = EW // BCH
NPS = 624
NTAIL = N - NPS * NS


@functools.partial(
    pl.kernel,
    out_type=jax.ShapeDtypeStruct((NC, N, D), jnp.float32),
    mesh=_mesh,
    scratch_types=[
        pltpu.VMEM_SHARED((N, D), jnp.float32),
        pltpu.VMEM((BCH,), jnp.int32),
        pltpu.VMEM((BCH,), jnp.int32),
        pltpu.VMEM((BCH, D), jnp.float32),
        pltpu.VMEM((BCH, D), jnp.float32),
        pltpu.VMEM((BCH,), jnp.int32),
        pltpu.VMEM((BCH,), jnp.int32),
        pltpu.VMEM((BCH, D), jnp.float32),
        pltpu.VMEM((BCH, D), jnp.float32),
        pltpu.SemaphoreType.DMA,
        pltpu.SemaphoreType.DMA,
        pltpu.SemaphoreType.DMA,
        pltpu.SemaphoreType.DMA,
        pltpu.SemaphoreType.DMA,
        pltpu.SemaphoreType.DMA,
    ],
    compiler_params=_sc_params,
)
def _sc_agg_kernel(src_hbm, dst_hbm, h_hbm, gate_hbm, agg_hbm,
                   aggS, src0, dst0, hrow0, gate0, src1, dst1, hrow1, gate1,
                   asem0, asem1, gsem0, gsem1, ssem0, ssem1):
    c = lax.axis_index("c")
    s = lax.axis_index("s")
    wid = s * NC + c
    bufs = ((src0, dst0, hrow0, gate0, asem0, gsem0, ssem0),
            (src1, dst1, hrow1, gate1, asem1, gsem1, ssem1))

    z16 = jnp.zeros((16,), jnp.float32)

    def _z(e, _):
        for k in range(D // 16):
            hrow0[e, pl.ds(k * 16, 16)] = z16
        return 0
    lax.fori_loop(0, BCH, _z, 0)
    for t in range(NPS // BCH):
        pltpu.sync_copy(hrow0, aggS.at[pl.ds(s * NPS + t * BCH, BCH)])
    rem = NPS - (NPS // BCH) * BCH
    pltpu.sync_copy(hrow0.at[pl.ds(0, rem)],
                    aggS.at[pl.ds(s * NPS + (NPS // BCH) * BCH, rem)])

    @pl.when(s == NS - 1)
    def _ztail():
        pltpu.sync_copy(hrow0.at[pl.ds(0, NTAIL)],
                        aggS.at[pl.ds(NS * NPS, NTAIL)])
    plsc.subcore_barrier()

    def _startA(b, ci):
        sv, dv, _, gv, asem, _, _ = bufs[b]
        e0 = wid * EW + ci * BCH
        pltpu.async_copy(src_hbm.at[pl.ds(e0, BCH)], sv, asem)
        pltpu.async_copy(dst_hbm.at[pl.ds(e0, BCH)], dv, asem)
        pltpu.async_copy(gate_hbm.at[pl.ds(e0, BCH)], gv, asem)

    def _waitA(b):
        sv, dv, _, gv, asem, _, _ = bufs[b]
        pltpu.make_async_copy(src_hbm.at[pl.ds(0, BCH)], sv, asem).wait()
        pltpu.make_async_copy(dst_hbm.at[pl.ds(0, BCH)], dv, asem).wait()
        pltpu.make_async_copy(gate_hbm.at[pl.ds(0, BCH)], gv, asem).wait()

    def _startG(b):
        sv, _, hv, _, _, gsem, _ = bufs[b]
        pltpu.async_copy(h_hbm.at[sv], hv, gsem)

    def _waitG(b):
        sv, _, hv, _, _, gsem, _ = bufs[b]
        pltpu.make_async_copy(h_hbm.at[sv], hv, gsem).wait()

    def _compute(b):
        _, dv, hv, gv, _, _, ssem = bufs[b]

        def _mul(e2, _):
            for u in range(2):
                for k in range(D // 16):
                    sl = pl.ds(k * 16, 16)
                    hv[e2 * 2 + u, sl] = hv[e2 * 2 + u, sl] * gv[e2 * 2 + u, sl]
            return 0
        lax.fori_loop(0, BCH // 2, _mul, 0)
        pltpu.async_copy(hv, aggS.at[dv], ssem, add=True)

    def _waitS(b):
        _, dv, hv, _, _, _, ssem = bufs[b]
        pltpu.make_async_copy(hv, aggS.at[dv], ssem).wait()

    _startA(0, 0)
    _startA(1, 1)
    _waitA(0)
    _startG(0)

    def _pair(p, _):
        for b in range(2):
            ci = 2 * p + b

            @pl.when(ci + 1 < BNCH)
            def _nxt():
                _waitA(1 - b)

                @pl.when(ci >= 1)
                def _ws():
                    _waitS(1 - b)
                _startG(1 - b)
            _waitG(b)
            _compute(b)

            @pl.when(ci + 2 < BNCH)
            def _pref():
                _startA(b, ci + 2)
        return 0
    lax.fori_loop(0, BNCH // 2, _pair, 0)

    _waitG(0)
    _compute(0)
    _waitS(1)
    _waitS(0)

    plsc.subcore_barrier()
    pltpu.sync_copy(aggS.at[pl.ds(s * NPS, NPS)],
                    agg_hbm.at[c, pl.ds(s * NPS, NPS)])

    @pl.when(s == NS - 1)
    def _wtail():
        pltpu.sync_copy(aggS.at[pl.ds(NS * NPS, NTAIL)],
                        agg_hbm.at[c, pl.ds(NS * NPS, NTAIL)])


def _tail_body(agg0_ref, agg1_ref, h_ref, batch_ref, gf_ref, wc_ref, bc_ref,
               o_ref, pooled_ref):
    i = pl.program_id(0)

    @pl.when(i == 0)
    def _init():
        pooled_ref[...] = jnp.zeros_like(pooled_ref)

    ge = jax.nn.relu(agg0_ref[0] + agg1_ref[0] + h_ref[...])
    b = batch_ref[0, 0, :]
    seg = lax.broadcasted_iota(jnp.int32, (G, NB), 0)
    onehot = (seg == b[None, :]).astype(jnp.float32)
    pooled_ref[...] += jnp.dot(onehot, ge, preferred_element_type=jnp.float32)

    @pl.when(i == NBLK - 1)
    def _final():
        gf_wc = jnp.dot(gf_ref[...], wc_ref[D:, :],
                        preferred_element_type=jnp.float32)
        o_ref[...] = (jnp.dot(pooled_ref[...], wc_ref[:D, :],
                              preferred_element_type=jnp.float32)
                      + gf_wc + bc_ref[...])


def _tail(agg2, h, batch, global_features, Wc, bc):
    batch3 = batch.reshape(NBLK, 1, NB)
    return pl.pallas_call(
        _tail_body,
        grid=(NBLK,),
        in_specs=[
            pl.BlockSpec((1, NB, D), lambda i: (0, i, 0)),
            pl.BlockSpec((1, NB, D), lambda i: (1, i, 0)),
            pl.BlockSpec((NB, D), lambda i: (i, 0)),
            pl.BlockSpec((1, 1, NB), lambda i: (i, 0, 0)),
            pl.BlockSpec((1, GF), lambda i: (0, 0)),
            pl.BlockSpec((D + GF, C), lambda i: (0, 0)),
            pl.BlockSpec((1, C), lambda i: (0, 0)),
        ],
        out_specs=pl.BlockSpec((G, C), lambda i: (0, 0)),
        out_shape=jax.ShapeDtypeStruct((G, C), jnp.float32),
        scratch_shapes=[pltpu.VMEM((G, D), jnp.float32)],
    )(agg2, agg2, h, batch3, global_features, Wc, bc.reshape(1, C))


def kernel(x, edge_index, edge_attr, batch, global_features, W1, b1, We, Wc, bc):
    src = edge_index[0]
    dst = edge_index[1]
    attr_flat = edge_attr.reshape(E * DE)

    eb, minp = _sc_min_kernel(src, attr_flat, batch)

    h = _compute_h(x, W1, b1)
    gate = _compute_gate(edge_attr, eb, minp, We)

    agg2 = _sc_agg_kernel(src, dst, h, gate)

    return _tail(agg2, h, batch, global_features, Wc, bc)
